# baseline (speedup 1.0000x reference)
import functools

import jax
import jax.numpy as jnp
from jax import lax
from jax.experimental import pallas as pl
from jax.experimental.pallas import tpu as pltpu
from jax.experimental.pallas import tpu_sc as plsc

B, L = 4096, 128
VOCAB, DIM = 100000, 64

NC, NS, LANES = 2, 16, 16
NW = NC * NS
SEQ_PER_W = B // NW
NH = DIM // 32
NBUF = 8
UNROLL = 8

_mesh = plsc.VectorSubcoreMesh(core_axis_name="c", subcore_axis_name="s")

def _unpack(vi):
    ev = plsc.bitcast(lax.shift_left(vi, 16), jnp.float32)
    od = plsc.bitcast(lax.bitwise_and(vi, jnp.int32(-65536)), jnp.float32)
    return ev, od


@functools.partial(
    pl.kernel,
    mesh=_mesh,
    out_type=jax.ShapeDtypeStruct((B, DIM), jnp.float32),
    scratch_types=[
        pltpu.VMEM((SEQ_PER_W * L,), jnp.int32),
        pltpu.VMEM((NBUF, L, DIM // 2), jnp.int32),
        pltpu.VMEM((1, DIM // 2), jnp.int32),
        pltpu.VMEM((SEQ_PER_W, DIM), jnp.float32),
    ] + [pltpu.SemaphoreType.DMA] * NBUF,
    compiler_params=pltpu.CompilerParams(
        use_tc_tiling_on_sc=False, needs_layout_passes=False),
)
def _embed_pool(ids_hbm, table_hbm, out_hbm, ids_v, rows_v, t0_v, out_v,
                *sems):
    wid = lax.axis_index("s") * NC + lax.axis_index("c")
    base = wid * SEQ_PER_W

    pltpu.sync_copy(ids_hbm.at[pl.ds(base * L, SEQ_PER_W * L)], ids_v)
    pltpu.sync_copy(table_hbm.at[pl.ds(0, 1)], t0_v)

    t0sep = []
    for h in range(NH):
        ev, od = _unpack(t0_v[0, pl.ds(h * LANES, LANES)])
        t0sep += [ev, od]

    lane = lax.iota(jnp.int32, LANES)
    even_lane = lax.rem(lane, 2) == 0
    half = lax.div(lane, 2)
    dnums = lax.GatherDimensionNumbers(
        offset_dims=(), collapsed_slice_dims=(0,), start_index_map=(0,))

    def perm(x, idx):
        return lax.gather(x, idx[:, None], dnums, (1,),
                          mode=lax.GatherScatterMode.PROMISE_IN_BOUNDS)

    def start(s, b):
        pltpu.async_copy(
            table_hbm.at[ids_v.at[pl.ds(s * L, L)]], rows_v.at[b], sems[b])

    def wait(s, b):
        pltpu.make_async_copy(
            table_hbm.at[ids_v.at[pl.ds(s * L, L)]], rows_v.at[b],
            sems[b]).wait()

    def process(s, b):
        rv = rows_v.at[b]

        def row_body(r, accs):
            new = []
            for u in range(UNROLL):
                au = list(accs[u * 2 * NH:(u + 1) * 2 * NH])
                for h in range(NH):
                    ev, od = _unpack(
                        rv[r * UNROLL + u, pl.ds(h * LANES, LANES)])
                    au[2 * h] = au[2 * h] + ev
                    au[2 * h + 1] = au[2 * h + 1] + od
                new.extend(au)
            return tuple(new)

        zero = jnp.zeros((LANES,), jnp.float32)
        accs = lax.fori_loop(0, L // UNROLL, row_body,
                             (zero,) * (UNROLL * 2 * NH))

        n0v = jnp.zeros((LANES,), jnp.int32)
        for k in range(L // LANES):
            n0v = n0v + jnp.where(
                ids_v[pl.ds(s * L + k * LANES, LANES)] == 0, 1, 0)
        for sh in (1, 2, 4, 8):
            n0v = n0v + perm(n0v, lane ^ sh)
        n0f = n0v.astype(jnp.float32)
        cnt = jnp.float32(L) - n0f

        for h in range(NH):
            tot_e = accs[2 * h]
            tot_o = accs[2 * h + 1]
            for u in range(1, UNROLL):
                tot_e = tot_e + accs[u * 2 * NH + 2 * h]
                tot_o = tot_o + accs[u * 2 * NH + 2 * h + 1]
            res_e = (tot_e - n0f * t0sep[2 * h]) / cnt
            res_o = (tot_o - n0f * t0sep[2 * h + 1]) / cnt
            lo = jnp.where(even_lane, perm(res_e, half), perm(res_o, half))
            hi = jnp.where(even_lane, perm(res_e, half + 8), perm(res_o, half + 8))
            out_v[s, pl.ds(h * 32, LANES)] = lo
            out_v[s, pl.ds(h * 32 + LANES, LANES)] = hi

    for b in range(NBUF):
        start(b, b)

    def group_body(g, carry):
        for b in range(NBUF):
            s = g * NBUF + b
            wait(s, b)
            process(s, b)

            @pl.when(s + NBUF < SEQ_PER_W)
            def _():
                start(s + NBUF, b)
        return carry

    lax.fori_loop(0, SEQ_PER_W // NBUF, group_body, 0)
    pltpu.sync_copy(out_v, out_hbm.at[pl.ds(base, SEQ_PER_W)])


def kernel(input_ids, table):
    ids = jnp.asarray(input_ids, jnp.int32).reshape(-1)
    tb = lax.bitcast_convert_type(
        table.astype(jnp.bfloat16).reshape(VOCAB, DIM // 2, 2), jnp.int32)
    return _embed_pool(ids, tb)


# speedup vs baseline: 2.3555x; 2.3555x over previous
import functools

import jax
import jax.numpy as jnp
from jax import lax
from jax.experimental import pallas as pl
from jax.experimental.pallas import tpu as pltpu
from jax.experimental.pallas import tpu_sc as plsc

B, L = 4096, 128
VOCAB, DIM = 100000, 64

NC, NS, LANES = 2, 16, 16
NW = NC * NS
SEQ_PER_W = B // NW
NV = DIM // LANES
NBUF = 8
UNROLL = 8

_mesh = plsc.VectorSubcoreMesh(core_axis_name="c", subcore_axis_name="s")


@functools.partial(
    pl.kernel,
    mesh=_mesh,
    out_type=jax.ShapeDtypeStruct((B, DIM), jnp.float32),
    scratch_types=[
        pltpu.VMEM((SEQ_PER_W * L,), jnp.int32),
        pltpu.VMEM((NBUF, L, DIM), jnp.float32),
        pltpu.VMEM((1, DIM), jnp.float32),
        pltpu.VMEM((SEQ_PER_W, DIM), jnp.float32),
    ] + [pltpu.SemaphoreType.DMA] * NBUF,
    compiler_params=pltpu.CompilerParams(use_tc_tiling_on_sc=False),
)
def _embed_pool(ids_hbm, table_hbm, out_hbm, ids_v, rows_v, t0_v, out_v,
                *sems):
    wid = lax.axis_index("s") * NC + lax.axis_index("c")
    base = wid * SEQ_PER_W

    pltpu.sync_copy(ids_hbm.at[pl.ds(base * L, SEQ_PER_W * L)], ids_v)
    pltpu.sync_copy(table_hbm.at[pl.ds(0, 1)], t0_v)

    def start(s, b):
        pltpu.async_copy(
            table_hbm.at[ids_v.at[pl.ds(s * L, L)]], rows_v.at[b], sems[b])

    def wait(s, b):
        pltpu.make_async_copy(
            table_hbm.at[ids_v.at[pl.ds(s * L, L)]], rows_v.at[b],
            sems[b]).wait()

    def process(s, b):
        rv = rows_v.at[b]

        def row_body(r, accs):
            new = []
            for u in range(UNROLL):
                au = list(accs[u * NV:(u + 1) * NV])
                for d in range(NV):
                    au[d] = au[d] + rv[r * UNROLL + u, pl.ds(d * LANES, LANES)]
                new.extend(au)
            return tuple(new)

        zero = jnp.zeros((LANES,), jnp.float32)
        accs = lax.fori_loop(0, L // UNROLL, row_body,
                             (zero,) * (UNROLL * NV))

        n0v = jnp.zeros((LANES,), jnp.int32)
        for k in range(L // LANES):
            n0v = n0v + jnp.where(
                ids_v[pl.ds(s * L + k * LANES, LANES)] == 0, 1, 0)
        lane = lax.iota(jnp.int32, LANES)
        dnums = lax.GatherDimensionNumbers(
            offset_dims=(), collapsed_slice_dims=(0,), start_index_map=(0,))
        for sh in (1, 2, 4, 8):
            perm = (lane ^ sh)[:, None]
            n0v = n0v + lax.gather(
                n0v, perm, dnums, (1,),
                mode=lax.GatherScatterMode.PROMISE_IN_BOUNDS)
        n0f = n0v.astype(jnp.float32)
        cnt = jnp.float32(L) - n0f

        for d in range(NV):
            tot = accs[d]
            for u in range(1, UNROLL):
                tot = tot + accs[u * NV + d]
            t0 = t0_v[0, pl.ds(d * LANES, LANES)]
            out_v[s, pl.ds(d * LANES, LANES)] = (tot - n0f * t0) / cnt

    for b in range(NBUF):
        start(b, b)

    def group_body(g, carry):
        for b in range(NBUF):
            s = g * NBUF + b
            wait(s, b)
            process(s, b)

            @pl.when(s + NBUF < SEQ_PER_W)
            def _():
                start(s + NBUF, b)
        return carry

    lax.fori_loop(0, SEQ_PER_W // NBUF, group_body, 0)
    pltpu.sync_copy(out_v, out_hbm.at[pl.ds(base, SEQ_PER_W)])


def kernel(input_ids, table):
    ids = jnp.asarray(input_ids, jnp.int32).reshape(-1)
    return _embed_pool(ids, table)
